# Initial kernel scaffold; baseline (speedup 1.0000x reference)
import jax
import jax.numpy as jnp
from jax.experimental import pallas as pl


def kernel(xyz1, xyz2, feat_points1, feat_points2, params):
    raise NotImplementedError("write your pallas kernel here")


# speedup vs baseline: 3.2363x; 3.2363x over previous
import functools

import jax
import jax.numpy as jnp
from jax.experimental import pallas as pl

SIGMA = 0.05
K = 16
BLK = 256
BIG = 1e30


def _dot(a, b, dims):
    return jax.lax.dot_general(a, b, dimension_numbers=(dims, ((), ())),
                               preferred_element_type=jnp.float32)


def _dotx(a, b, dims):
    return jax.lax.dot_general(a, b, dimension_numbers=(dims, ((), ())),
                               preferred_element_type=jnp.float32,
                               precision=jax.lax.Precision.HIGHEST)


def _knn_kernel(x1blk_ref, x1full_ref, x2_ref, f2_ref,
                s1blk_ref, s1row_ref, s2row_ref,
                interp_ref, dens_ref, idxk_ref, gxyz_ref):
    x1b = x1blk_ref[0]
    x1f = x1full_ref[0]
    x2 = x2_ref[0]
    f2 = f2_ref[0]

    s1b = s1blk_ref[0]
    s2 = s2row_ref[0]
    d2 = (s1b + s2) - 2.0 * _dot(x1b, x2, ((1,), (1,)))
    n2 = d2.shape[1]
    iota2 = jax.lax.broadcasted_iota(jnp.int32, d2.shape, 1)
    wsel = jnp.zeros_like(d2)
    wsum = jnp.zeros((d2.shape[0], 1), jnp.float32)
    for _ in range(3):
        m = jnp.min(d2, axis=1, keepdims=True)
        am = jnp.min(jnp.where(d2 == m, iota2, n2), axis=1, keepdims=True)
        sel = (iota2 == am)
        wv = 1.0 / jnp.maximum(m, 1e-10)
        wsel = wsel + jnp.where(sel, wv, 0.0)
        wsum = wsum + wv
        d2 = jnp.where(sel, BIG, d2)
    interp_ref[0] = _dotx(wsel / wsum, f2, ((1,), (0,)))

    s1f = s1row_ref[0]
    d11 = (s1b + s1f) - 2.0 * _dot(x1b, x1f, ((1,), (1,)))
    n1 = d11.shape[1]
    inv2s2 = 1.0 / (2.0 * SIGMA * SIGMA)
    gauss = jnp.exp(-d11 * inv2s2)
    dens_ref[0] = jnp.sum(gauss, axis=1)[None, :] * (1.0 / (n1 * 2.5 * SIGMA))

    iota1 = jax.lax.broadcasted_iota(jnp.int32, d11.shape, 1)
    for k in range(K):
        m = jnp.min(d11, axis=1, keepdims=True)
        am = jnp.min(jnp.where(d11 == m, iota1, n1), axis=1, keepdims=True)
        sel = (iota1 == am)
        idxk_ref[0, :, k:k + 1] = am
        g = _dotx(sel.astype(jnp.float32), x1f, ((1,), (0,)))
        gxyz_ref[0, :, k, :] = g - x1b
        d11 = jnp.where(sel, BIG, d11)


def _densitynet_kernel(x_ref, w0_ref, b0_ref, g0_ref, e0_ref,
                       w1_ref, b1_ref, g1_ref, e1_ref,
                       w2_ref, b2_ref, g2_ref, e2_ref, out_ref):
    x = x_ref[...]
    h = x
    for w_r, b_r, g_r, e_r, first in ((w0_ref, b0_ref, g0_ref, e0_ref, True),
                                      (w1_ref, b1_ref, g1_ref, e1_ref, False),
                                      (w2_ref, b2_ref, g2_ref, e2_ref, False)):
        if first:
            h = w_r[...] * x + b_r[...]
        else:
            h = _dot(w_r[...], h, ((1,), (0,))) + b_r[...]
        mu = jnp.mean(h, axis=1, keepdims=True)
        var = jnp.mean(h * h, axis=1, keepdims=True) - mu * mu
        h = (h - mu) * jax.lax.rsqrt(var + 1e-5) * g_r[...] + e_r[...]
        h = jnp.maximum(h, 0.0)
    out_ref[...] = h


def _weightnet_kernel(x_ref, w0_ref, b0_ref, g0_ref, e0_ref,
                      w1_ref, b1_ref, g1_ref, e1_ref,
                      w2_ref, b2_ref, g2_ref, e2_ref, out_ref):
    h = x_ref[...]
    for w_r, b_r, g_r, e_r in ((w0_ref, b0_ref, g0_ref, e0_ref),
                               (w1_ref, b1_ref, g1_ref, e1_ref),
                               (w2_ref, b2_ref, g2_ref, e2_ref)):
        h = _dot(w_r[...], h, ((1,), (0,))) + b_r[...]
        mu = jnp.mean(h, axis=1, keepdims=True)
        var = jnp.mean(h * h, axis=1, keepdims=True) - mu * mu
        h = (h - mu) * jax.lax.rsqrt(var + 1e-5) * g_r[...] + e_r[...]
        h = jnp.maximum(h, 0.0)
    out_ref[...] = h


def _aggregate_kernel(interp_ref, ds_ref, idxk_ref, wn_ref, c0w_ref, c0b_ref,
                      y0_ref, s1_ref, s2_ref):
    fused = interp_ref[0] * ds_ref[0]
    n1 = fused.shape[0]
    blk = idxk_ref.shape[1]
    iota = jax.lax.broadcasted_iota(jnp.int32, (blk, n1), 1)
    acc = jnp.zeros((blk, K, 64), jnp.float32)
    for k in range(K):
        idxcol = idxk_ref[0, :, k:k + 1]
        sel = (iota == idxcol).astype(jnp.float32)
        g = _dotx(sel, fused, ((1,), (0,)))
        wk = wn_ref[0, :, k, :]
        acc = acc + wk[:, :, None] * g[:, None, :]
    y = _dot(acc.reshape(blk * K, 64), c0w_ref[...], ((1,), (1,))) + c0b_ref[...]
    y0_ref[0] = y.reshape(blk, K, 64)
    s1_ref[0] = jnp.sum(y, axis=0, keepdims=True)
    s2_ref[0] = jnp.sum(y * y, axis=0, keepdims=True)


def _m0_kernel(y0_ref, s1_ref, s2_ref, g_ref, e_ref, f1_ref,
               m0a_ref, m0b_ref, m0bias_ref, z_ref, *, count):
    s1 = jnp.sum(s1_ref[:, 0, :], axis=0, keepdims=True)
    s2 = jnp.sum(s2_ref[:, 0, :], axis=0, keepdims=True)
    mu = s1 / count
    var = s2 / count - mu * mu
    scale = jax.lax.rsqrt(var + 1e-5) * g_ref[...]
    shift = e_ref[...] - mu * scale
    y = y0_ref[0]
    blk = y.shape[0]
    h = jnp.maximum(y * scale[None] + shift[None], 0.0).reshape(blk, K * 64)
    z = (_dot(h, m0a_ref[...], ((1,), (0,)))
         + _dot(f1_ref[0], m0b_ref[...], ((1,), (0,))) + m0bias_ref[...])
    z_ref[0] = z


def _head_kernel(z_ref, g0_ref, e0_ref, w1_ref, b1_ref, g1_ref, e1_ref,
                 out_ref):
    z = z_ref[...]
    for first in (True, False):
        if first:
            g_r, e_r = g0_ref, e0_ref
        else:
            z = _dot(z, w1_ref[...], ((1,), (0,))) + b1_ref[...]
            g_r, e_r = g1_ref, e1_ref
        mu = jnp.mean(z, axis=0, keepdims=True)
        var = jnp.mean(z * z, axis=0, keepdims=True) - mu * mu
        z = (z - mu) * jax.lax.rsqrt(var + 1e-5) * g_r[...] + e_r[...]
        z = jnp.maximum(z, 0.0)
    out_ref[...] = z


def kernel(xyz1, xyz2, feat_points1, feat_points2, params):
    p = params
    B, _, N1 = xyz1.shape
    N2 = xyz2.shape[2]
    C2 = feat_points2.shape[1]
    NB = N1 // BLK
    f32 = jnp.float32

    x1t = jnp.transpose(xyz1, (0, 2, 1))
    x2t = jnp.transpose(xyz2, (0, 2, 1))
    f2t = jnp.transpose(feat_points2, (0, 2, 1))
    f1t = jnp.transpose(feat_points1, (0, 2, 1))

    grid = (B, NB)
    interp, dens, idxk, gxyz = pl.pallas_call(
        _knn_kernel,
        grid=grid,
        in_specs=[
            pl.BlockSpec((1, BLK, 3), lambda b, i: (b, i, 0)),
            pl.BlockSpec((1, N1, 3), lambda b, i: (b, 0, 0)),
            pl.BlockSpec((1, N2, 3), lambda b, i: (b, 0, 0)),
            pl.BlockSpec((1, N2, C2), lambda b, i: (b, 0, 0)),
            pl.BlockSpec((1, BLK, 1), lambda b, i: (b, i, 0)),
            pl.BlockSpec((1, 1, N1), lambda b, i: (b, 0, 0)),
            pl.BlockSpec((1, 1, N2), lambda b, i: (b, 0, 0)),
        ],
        out_specs=[
            pl.BlockSpec((1, BLK, C2), lambda b, i: (b, i, 0)),
            pl.BlockSpec((1, 1, BLK), lambda b, i: (b * NB + i, 0, 0)),
            pl.BlockSpec((1, BLK, K), lambda b, i: (b, i, 0)),
            pl.BlockSpec((1, BLK, K, 3), lambda b, i: (b, i, 0, 0)),
        ],
        out_shape=[
            jax.ShapeDtypeStruct((B, N1, C2), f32),
            jax.ShapeDtypeStruct((B * NB, 1, BLK), f32),
            jax.ShapeDtypeStruct((B, N1, K), jnp.int32),
            jax.ShapeDtypeStruct((B, N1, K, 3), f32),
        ],
    )(x1t, x1t, x2t, f2t,
      jnp.sum(x1t * x1t, -1)[:, :, None],
      jnp.sum(x1t * x1t, -1)[:, None, :],
      jnp.sum(x2t * x2t, -1)[:, None, :])

    M = B * N1
    dens_flat = dens.reshape(1, M)

    def vec(a, rows):
        return a.reshape(rows, 1) if a.ndim == 1 else a

    dn_args = []
    for i in range(3):
        o = p['dn%d_w' % i].shape[0]
        dn_args += [p['dn%d_w' % i], vec(p['dn%d_b' % i], o),
                    vec(p['dn%d_g' % i], o), vec(p['dn%d_be' % i], o)]
    def full(shape):
        return pl.BlockSpec(shape, lambda *a, _n=len(shape): (0,) * _n)
    ds = pl.pallas_call(
        _densitynet_kernel,
        in_specs=[full((1, M))] + [full(a.shape) for a in dn_args],
        out_specs=full((1, M)),
        out_shape=jax.ShapeDtypeStruct((1, M), f32),
    )(dens_flat, *dn_args)
    ds = ds.reshape(B, N1, 1)

    MK = B * N1 * K
    gx_flat = gxyz.reshape(MK, 3).T
    wn_args = []
    for i in range(3):
        o = p['wn%d_w' % i].shape[0]
        wn_args += [p['wn%d_w' % i], vec(p['wn%d_b' % i], o),
                    vec(p['wn%d_g' % i], o), vec(p['wn%d_be' % i], o)]
    wn = pl.pallas_call(
        _weightnet_kernel,
        in_specs=[full((3, MK))] + [full(a.shape) for a in wn_args],
        out_specs=full((16, MK)),
        out_shape=jax.ShapeDtypeStruct((16, MK), f32),
    )(gx_flat, *wn_args)
    wn = wn.reshape(16, B, N1, K).transpose(1, 2, 3, 0)

    y0, s1, s2 = pl.pallas_call(
        _aggregate_kernel,
        grid=grid,
        in_specs=[
            pl.BlockSpec((1, N1, C2), lambda b, i: (b, 0, 0)),
            pl.BlockSpec((1, N1, 1), lambda b, i: (b, 0, 0)),
            pl.BlockSpec((1, BLK, K), lambda b, i: (b, i, 0)),
            pl.BlockSpec((1, BLK, K, 16), lambda b, i: (b, i, 0, 0)),
            full((64, C2)),
            full((1, 64)),
        ],
        out_specs=[
            pl.BlockSpec((1, BLK, K, 64), lambda b, i: (b, i, 0, 0)),
            pl.BlockSpec((1, 1, 64), lambda b, i: (b * NB + i, 0, 0)),
            pl.BlockSpec((1, 1, 64), lambda b, i: (b * NB + i, 0, 0)),
        ],
        out_shape=[
            jax.ShapeDtypeStruct((B, N1, K, 64), f32),
            jax.ShapeDtypeStruct((B * NB, 1, 64), f32),
            jax.ShapeDtypeStruct((B * NB, 1, 64), f32),
        ],
    )(interp, ds, idxk, wn, p['c0_w'], p['c0_b'].reshape(1, 64))

    m0a = p['m0_w'][:, :K * 64].T
    m0b = p['m0_w'][:, K * 64:].T
    z0 = pl.pallas_call(
        functools.partial(_m0_kernel, count=float(B * N1 * K)),
        grid=grid,
        in_specs=[
            pl.BlockSpec((1, BLK, K, 64), lambda b, i: (b, i, 0, 0)),
            pl.BlockSpec((B * NB, 1, 64), lambda b, i: (0, 0, 0)),
            pl.BlockSpec((B * NB, 1, 64), lambda b, i: (0, 0, 0)),
            full((1, 64)),
            full((1, 64)),
            pl.BlockSpec((1, BLK, 64), lambda b, i: (b, i, 0)),
            full((K * 64, 64)),
            full((64, 64)),
            full((1, 64)),
        ],
        out_specs=pl.BlockSpec((1, BLK, 64), lambda b, i: (b, i, 0)),
        out_shape=jax.ShapeDtypeStruct((B, N1, 64), f32),
    )(y0, s1, s2, p['c0_g'].reshape(1, 64), p['c0_be'].reshape(1, 64),
      f1t, m0a, m0b, p['m0_b'].reshape(1, 64))

    z0f = z0.reshape(M, 64)
    out = pl.pallas_call(
        _head_kernel,
        in_specs=[full((M, 64)), full((1, 64)), full((1, 64)),
                  full((64, 64)), full((1, 64)), full((1, 64)), full((1, 64))],
        out_specs=full((M, 64)),
        out_shape=jax.ShapeDtypeStruct((M, 64), f32),
    )(z0f, p['m0_g'].reshape(1, 64), p['m0_be'].reshape(1, 64),
      p['m1_w'].T, p['m1_b'].reshape(1, 64),
      p['m1_g'].reshape(1, 64), p['m1_be'].reshape(1, 64))
    return out.reshape(B, N1, 64)
